# baseline (speedup 1.0000x reference)
import functools

import jax
import jax.numpy as jnp
from jax import lax
from jax.experimental import pallas as pl
from jax.experimental.pallas import tpu as pltpu
from jax.experimental.pallas import tpu_sc as plsc

NG = 50
NPG = 1000
NPAD = 1024
EPG = 16000
NN = NG * NPG
NE = NG * EPG
HID = 16
HEADS = 4
D = HID * HEADS
CHUNK = 640
NCH = EPG // CHUNK
GROUPS = CHUNK // 16
NWORK = 32
NUNITS = NG * HEADS
UPW = -(-NUNITS // NWORK)
TBL = 16 * NPAD

_f32 = jnp.float32
_i32 = jnp.int32


def _erf(x):
    s = jnp.sign(x)
    z = jnp.abs(x)
    t = 1.0 / (1.0 + 0.3275911 * z)
    poly = t * (0.254829592 + t * (-0.284496736 + t * (1.421413741
           + t * (-1.453152027 + t * 1.061405429))))
    return s * (1.0 - poly * jnp.exp(-z * z))


def _gelu(x):
    return 0.5 * x * (1.0 + _erf(x * 0.7071067811865476))


def _dot(a, b):
    return lax.dot_general(a, b, (((1,), (0,)), ((), ())),
                           preferred_element_type=_f32)


def _node_proj_body(xT_ref, wni_ref, wnj_ref, wnd_ref, ni_ref, nj_ref, nd_ref):
    x = xT_ref[0]
    ni_ref[0] = _dot(wni_ref[...], x)
    nj_ref[0] = _dot(wnj_ref[...], x)
    nd_ref[0] = _dot(wnd_ref[...], x)


def _node_proj(xT, wniT, wnjT, wndT):
    k = xT.shape[1]
    out = jax.ShapeDtypeStruct((NG, D, NPAD), _f32)
    return pl.pallas_call(
        _node_proj_body,
        grid=(NG,),
        in_specs=[
            pl.BlockSpec((1, k, NPAD), lambda g: (g, 0, 0)),
            pl.BlockSpec((D, k), lambda g: (0, 0)),
            pl.BlockSpec((D, k), lambda g: (0, 0)),
            pl.BlockSpec((D, k), lambda g: (0, 0)),
        ],
        out_specs=[pl.BlockSpec((1, D, NPAD), lambda g: (g, 0, 0))] * 3,
        out_shape=[out, out, out],
    )(xT, wniT, wnjT, wndT)


def _edge_proj_body(xT_ref, wf_ref, b_ref, out_ref):
    x = xT_ref[0]
    out_ref[0] = _dot(wf_ref[...], x) + b_ref[...]


def _edge_proj(xT, wfT, biasT):
    k = xT.shape[1]
    return pl.pallas_call(
        _edge_proj_body,
        grid=(NG,),
        in_specs=[
            pl.BlockSpec((1, k, EPG), lambda g: (g, 0, 0)),
            pl.BlockSpec((D, k), lambda g: (0, 0)),
            pl.BlockSpec((D, 1), lambda g: (0, 0)),
        ],
        out_specs=pl.BlockSpec((1, D, EPG), lambda g: (g, 0, 0)),
        out_shape=jax.ShapeDtypeStruct((NG, D, EPG), _f32),
    )(xT, wfT, biasT)


def _readout_body(mask_cols, xT_ref, w_ref, b_ref, out_ref):
    n = xT_ref.shape[2]
    y = _gelu(_dot(w_ref[...], xT_ref[0]) + b_ref[...])
    if mask_cols is not None:
        col = lax.broadcasted_iota(_i32, (D, n), 1)
        y = jnp.where(col < mask_cols, y, 0.0)
    out_ref[0, 0] = jnp.sum(y, axis=1)


def _readout(xT, wT, biasT, n, mask_cols=None):
    return pl.pallas_call(
        functools.partial(_readout_body, mask_cols),
        grid=(NG,),
        in_specs=[
            pl.BlockSpec((1, D, n), lambda g: (g, 0, 0)),
            pl.BlockSpec((D, D), lambda g: (0, 0)),
            pl.BlockSpec((D, 1), lambda g: (0, 0)),
        ],
        out_specs=pl.BlockSpec((1, 1, D), lambda g: (g, 0, 0)),
        out_shape=jax.ShapeDtypeStruct((NG, 1, D), _f32),
    )(xT, wT, biasT)


def _mlp_body(g_ref, w1, b1, w2, b2, w3, b3, out_ref):
    x = g_ref[...]
    x = _gelu(x @ w1[...] + b1[...][None, :])
    x = _gelu(x @ w2[...] + b2[...][None, :])
    out_ref[...] = x @ w3[...] + b3[...][None, :]


_sc_mesh = plsc.VectorSubcoreMesh(core_axis_name="c", subcore_axis_name="s")

_BCAST_DNUMS = lax.GatherDimensionNumbers(
    offset_dims=(), collapsed_slice_dims=(0,), start_index_map=(0,))


def _bcast(vec, l):
    return lax.gather(vec, jnp.full((16, 1), l, _i32), _BCAST_DNUMS, (1,),
                      mode=lax.GatherScatterMode.PROMISE_IN_BOUNDS)


@functools.partial(
    pl.kernel,
    mesh=_sc_mesh,
    compiler_params=pltpu.CompilerParams(needs_layout_passes=False),
    out_type=(
        jax.ShapeDtypeStruct((NG, D, EPG), _f32),
        jax.ShapeDtypeStruct((NG * D * NPAD,), _f32),
    ),
    scratch_types=[
        pltpu.VMEM((TBL,), _f32),
        pltpu.VMEM((TBL,), _f32),
        pltpu.VMEM((EPG,), _i32),
        pltpu.VMEM((EPG,), _i32),
        pltpu.VMEM((EPG,), _f32),
        pltpu.VMEM((16 * NPAD,), _f32),
        pltpu.VMEM((TBL,), _f32),
        pltpu.VMEM((16, CHUNK), _f32),
        pltpu.VMEM((D,), _f32),
        pltpu.VMEM((16 * 16,), _f32),
    ],
)
def _sc_egat(niT, njT, hT, fijT, attn, src, dst, foutT, haccT,
             tbl_a, tbl_b, src_sp, dst_sp, ee_sp, den16_sp, hacc_sp,
             buf, attn_sp, attn_bc):
    wid = lax.axis_index("c") * 16 + lax.axis_index("s")
    pltpu.sync_copy(attn, attn_sp)
    zero16 = jnp.zeros((16,), _f32)
    lane = lax.iota(_i32, 16)
    lane_bank = lane * NPAD

    def unit_body(k, carry):
        uid = wid + k * NWORK

        @pl.when(uid < NUNITS)
        def _():
            g = uid // HEADS
            h = uid % HEADS
            tb = (g * D + h * 16) * NPAD
            pltpu.sync_copy(niT.at[pl.ds(tb, TBL)], tbl_a)
            pltpu.sync_copy(njT.at[pl.ds(tb, TBL)], tbl_b)
            pltpu.sync_copy(src.at[pl.ds(g * EPG, EPG)], src_sp)
            pltpu.sync_copy(dst.at[pl.ds(g * EPG, EPG)], dst_sp)
            hvec = jnp.full((16,), h * 16, _i32)
            for j in range(16):
                attn_bc[pl.ds(j * 16, 16)] = plsc.load_gather(
                    attn_sp, [hvec + j])

            def zero_body(i, c):
                for j in range(16):
                    hacc_sp[pl.ds(j * NPAD + i * 16, 16)] = zero16
                    den16_sp[pl.ds(j * NPAD + i * 16, 16)] = zero16
                return c
            lax.fori_loop(0, NPAD // 16, zero_body, 0)

            def chunk_body(c, cc):
                row0 = h * 16
                pltpu.sync_copy(
                    fijT.at[g, pl.ds(row0, 16), pl.ds(c * CHUNK, CHUNK)], buf)

                @plsc.parallel_loop(0, GROUPS)
                def group_body(gi):
                    base = c * CHUNK + gi * 16
                    src_v = src_sp[pl.ds(base, 16)]
                    dst_v = dst_sp[pl.ds(base, 16)]
                    eacc = [zero16] * 4
                    for j in range(16):
                        ni = plsc.load_gather(tbl_a, [src_v + j * NPAD])
                        nj = plsc.load_gather(tbl_b, [dst_v + j * NPAD])
                        s = ni + nj + buf[j, pl.ds(gi * 16, 16)]
                        f = jnp.maximum(s, s * 0.01)
                        buf[j, pl.ds(gi * 16, 16)] = f
                        eacc[j % 4] = eacc[j % 4] + f * attn_bc[pl.ds(j * 16, 16)]
                    ee = jnp.exp((eacc[0] + eacc[1]) + (eacc[2] + eacc[3]))
                    ee_sp[pl.ds(base, 16)] = ee
                    plsc.addupdate_scatter(den16_sp, [dst_v + lane_bank], ee)
                pltpu.sync_copy(
                    buf, foutT.at[g, pl.ds(row0, 16), pl.ds(c * CHUNK, CHUNK)])
                return cc
            lax.fori_loop(0, NCH, chunk_body, 0)

            pltpu.sync_copy(hT.at[pl.ds(tb, TBL)], tbl_a)

            @plsc.parallel_loop(0, EPG // 16, unroll=2)
            def p2_body(gi):
                base = gi * 16
                src_v = src_sp[pl.ds(base, 16)]
                dst_v = dst_sp[pl.ds(base, 16)]
                dacc = [zero16] * 4
                for b in range(16):
                    dacc[b % 4] = dacc[b % 4] + plsc.load_gather(
                        den16_sp, [dst_v + b * NPAD])
                den_v = (dacc[0] + dacc[1]) + (dacc[2] + dacc[3])
                a_v = ee_sp[pl.ds(base, 16)] / den_v
                for l in range(16):
                    src_b = _bcast(src_v, l)
                    dst_b = _bcast(dst_v, l)
                    a_b = _bcast(a_v, l)
                    hv = plsc.load_gather(tbl_a, [src_b + lane_bank])
                    plsc.addupdate_scatter(
                        hacc_sp, [dst_b + lane_bank], hv * a_b)
            pltpu.sync_copy(hacc_sp, haccT.at[pl.ds(tb, TBL)])
        return carry

    lax.fori_loop(0, UPW, unit_body, 0)


def kernel(node_x, edge_x, Radd, params, src, dst, node_gid, edge_gid):
    p1, p2 = params['egat1'], params['egat2']
    off = (jnp.arange(NE, dtype=_i32) // EPG) * NPG
    src_l = src - off
    dst_l = dst - off

    xT = node_x.reshape(NG, NPG, -1).transpose(0, 2, 1)
    xT = jnp.pad(xT, ((0, 0), (0, 0), (0, NPAD - NPG)))
    eT = edge_x.reshape(NG, EPG, -1).transpose(0, 2, 1)

    feats_n, feats_e = xT, eT
    for p in (p1, p2, p2):
        ni, nj, hh = _node_proj(feats_n, p['W_ni'].T, p['W_nj'].T,
                                p['W_node'].T)
        fij = _edge_proj(feats_e, p['W_fij'].T, p['bias'][:, None])
        foutT, hacc_flat = _sc_egat(
            ni.reshape(-1), nj.reshape(-1), hh.reshape(-1), fij,
            p['attn'].reshape(-1), src_l, dst_l)
        feats_n, feats_e = hacc_flat.reshape(NG, D, NPAD), foutT

    Gn = _readout(feats_n, params['aggN_W'].T, params['aggN_b'][:, None],
                  NPAD, mask_cols=NPG)
    Ge = _readout(feats_e, params['aggE_W'].T, params['aggE_b'][:, None], EPG)
    G = jnp.concatenate([Gn.reshape(NG, D), Ge.reshape(NG, D), Radd], axis=1)
    out = pl.pallas_call(
        _mlp_body,
        out_shape=jax.ShapeDtypeStruct((NG, 1), _f32),
    )(G, params['mlp1_W'], params['mlp1_b'], params['mlp2_W'],
      params['mlp2_b'], params['mlp3_W'], params['mlp3_b'])
    return out


# speedup vs baseline: 1.2035x; 1.2035x over previous
import functools

import jax
import jax.numpy as jnp
from jax import lax
from jax.experimental import pallas as pl
from jax.experimental.pallas import tpu as pltpu
from jax.experimental.pallas import tpu_sc as plsc

NG = 50
NPG = 1000
NPAD = 1024
EPG = 16000
NN = NG * NPG
NE = NG * EPG
HID = 16
HEADS = 4
D = HID * HEADS
CHUNK = 640
NCH = EPG // CHUNK
GROUPS = CHUNK // 16
NWORK = 32
NUNITS = NG * HEADS
UPW = -(-NUNITS // NWORK)
TBL = 16 * NPAD

_f32 = jnp.float32
_i32 = jnp.int32


def _erf(x):
    s = jnp.sign(x)
    z = jnp.abs(x)
    t = 1.0 / (1.0 + 0.3275911 * z)
    poly = t * (0.254829592 + t * (-0.284496736 + t * (1.421413741
           + t * (-1.453152027 + t * 1.061405429))))
    return s * (1.0 - poly * jnp.exp(-z * z))


def _gelu(x):
    return 0.5 * x * (1.0 + _erf(x * 0.7071067811865476))


def _dot(a, b):
    return lax.dot_general(a, b, (((1,), (0,)), ((), ())),
                           preferred_element_type=_f32)


def _node_proj_body(xT_ref, wni_ref, wnj_ref, wnd_ref, ni_ref, nj_ref, nd_ref):
    x = xT_ref[0]
    ni_ref[0] = _dot(wni_ref[...], x)
    nj_ref[0] = _dot(wnj_ref[...], x)
    nd_ref[0] = _dot(wnd_ref[...], x)


def _node_proj(xT, wniT, wnjT, wndT):
    k = xT.shape[1]
    out = jax.ShapeDtypeStruct((NG, D, NPAD), _f32)
    return pl.pallas_call(
        _node_proj_body,
        grid=(NG,),
        in_specs=[
            pl.BlockSpec((1, k, NPAD), lambda g: (g, 0, 0)),
            pl.BlockSpec((D, k), lambda g: (0, 0)),
            pl.BlockSpec((D, k), lambda g: (0, 0)),
            pl.BlockSpec((D, k), lambda g: (0, 0)),
        ],
        out_specs=[pl.BlockSpec((1, D, NPAD), lambda g: (g, 0, 0))] * 3,
        out_shape=[out, out, out],
    )(xT, wniT, wnjT, wndT)


def _edge_proj_body(xT_ref, wf_ref, b_ref, out_ref):
    x = xT_ref[0]
    out_ref[0] = _dot(wf_ref[...], x) + b_ref[...]


def _edge_proj(xT, wfT, biasT):
    k = xT.shape[1]
    return pl.pallas_call(
        _edge_proj_body,
        grid=(NG,),
        in_specs=[
            pl.BlockSpec((1, k, EPG), lambda g: (g, 0, 0)),
            pl.BlockSpec((D, k), lambda g: (0, 0)),
            pl.BlockSpec((D, 1), lambda g: (0, 0)),
        ],
        out_specs=pl.BlockSpec((1, D, EPG), lambda g: (g, 0, 0)),
        out_shape=jax.ShapeDtypeStruct((NG, D, EPG), _f32),
    )(xT, wfT, biasT)


def _readout_body(mask_cols, xT_ref, w_ref, b_ref, out_ref):
    n = xT_ref.shape[2]
    y = _gelu(_dot(w_ref[...], xT_ref[0]) + b_ref[...])
    if mask_cols is not None:
        col = lax.broadcasted_iota(_i32, (D, n), 1)
        y = jnp.where(col < mask_cols, y, 0.0)
    out_ref[0, 0] = jnp.sum(y, axis=1)


def _readout(xT, wT, biasT, n, mask_cols=None):
    return pl.pallas_call(
        functools.partial(_readout_body, mask_cols),
        grid=(NG,),
        in_specs=[
            pl.BlockSpec((1, D, n), lambda g: (g, 0, 0)),
            pl.BlockSpec((D, D), lambda g: (0, 0)),
            pl.BlockSpec((D, 1), lambda g: (0, 0)),
        ],
        out_specs=pl.BlockSpec((1, 1, D), lambda g: (g, 0, 0)),
        out_shape=jax.ShapeDtypeStruct((NG, 1, D), _f32),
    )(xT, wT, biasT)


def _mlp_body(g_ref, w1, b1, w2, b2, w3, b3, out_ref):
    x = g_ref[...]
    x = _gelu(x @ w1[...] + b1[...][None, :])
    x = _gelu(x @ w2[...] + b2[...][None, :])
    out_ref[...] = x @ w3[...] + b3[...][None, :]


_sc_mesh = plsc.VectorSubcoreMesh(core_axis_name="c", subcore_axis_name="s")

_BCAST_DNUMS = lax.GatherDimensionNumbers(
    offset_dims=(), collapsed_slice_dims=(0,), start_index_map=(0,))


def _bcast(vec, l):
    return lax.gather(vec, jnp.full((16, 1), l, _i32), _BCAST_DNUMS, (1,),
                      mode=lax.GatherScatterMode.PROMISE_IN_BOUNDS)


@functools.partial(
    pl.kernel,
    mesh=_sc_mesh,
    compiler_params=pltpu.CompilerParams(needs_layout_passes=False),
    out_type=(
        jax.ShapeDtypeStruct((NG, D, EPG), _f32),
        jax.ShapeDtypeStruct((NG * D * NPAD,), _f32),
    ),
    scratch_types=[
        pltpu.VMEM((TBL,), _f32),
        pltpu.VMEM((TBL,), _f32),
        pltpu.VMEM((EPG,), _i32),
        pltpu.VMEM((EPG,), _i32),
        pltpu.VMEM((EPG,), _f32),
        pltpu.VMEM((16 * NPAD,), _f32),
        pltpu.VMEM((TBL,), _f32),
        pltpu.VMEM((16, CHUNK), _f32),
        pltpu.VMEM((D,), _f32),
        pltpu.VMEM((16 * 16,), _f32),
    ],
)
def _sc_egat(niT, njT, hT, fijT, attn, src, dst, foutT, haccT,
             tbl_a, tbl_b, src_sp, dst_sp, ee_sp, den16_sp, hacc_sp,
             buf, attn_sp, attn_bc):
    wid = lax.axis_index("c") * 16 + lax.axis_index("s")
    pltpu.sync_copy(attn, attn_sp)
    zero16 = jnp.zeros((16,), _f32)
    lane = lax.iota(_i32, 16)
    lane_bank = lane * NPAD

    def unit_body(k, carry):
        uid = wid + k * NWORK

        @pl.when(uid < NUNITS)
        def _():
            g = uid // HEADS
            h = uid % HEADS
            tb = (g * D + h * 16) * NPAD
            pltpu.sync_copy(niT.at[pl.ds(tb, TBL)], tbl_a)
            pltpu.sync_copy(njT.at[pl.ds(tb, TBL)], tbl_b)
            pltpu.sync_copy(src.at[pl.ds(g * EPG, EPG)], src_sp)
            pltpu.sync_copy(dst.at[pl.ds(g * EPG, EPG)], dst_sp)
            hvec = jnp.full((16,), h * 16, _i32)
            for j in range(16):
                attn_bc[pl.ds(j * 16, 16)] = plsc.load_gather(
                    attn_sp, [hvec + j])

            def zero_body(i, c):
                for j in range(16):
                    hacc_sp[pl.ds(j * NPAD + i * 16, 16)] = zero16
                    den16_sp[pl.ds(j * NPAD + i * 16, 16)] = zero16
                return c
            lax.fori_loop(0, NPAD // 16, zero_body, 0)

            def chunk_body(c, cc):
                row0 = h * 16
                pltpu.sync_copy(
                    fijT.at[g, pl.ds(row0, 16), pl.ds(c * CHUNK, CHUNK)], buf)

                @plsc.parallel_loop(0, GROUPS)
                def group_body(gi):
                    base = c * CHUNK + gi * 16
                    src_v = src_sp[pl.ds(base, 16)]
                    dst_v = dst_sp[pl.ds(base, 16)]
                    eacc = [zero16] * 4
                    for j in range(16):
                        ni = plsc.load_gather(tbl_a, [src_v + j * NPAD])
                        nj = plsc.load_gather(tbl_b, [dst_v + j * NPAD])
                        s = ni + nj + buf[j, pl.ds(gi * 16, 16)]
                        f = jnp.maximum(s, s * 0.01)
                        buf[j, pl.ds(gi * 16, 16)] = f
                        eacc[j % 4] = eacc[j % 4] + f * attn_bc[pl.ds(j * 16, 16)]
                    ee = jnp.exp((eacc[0] + eacc[1]) + (eacc[2] + eacc[3]))
                    ee_sp[pl.ds(base, 16)] = ee
                    plsc.addupdate_scatter(den16_sp, [dst_v + lane_bank], ee)
                pltpu.sync_copy(
                    buf, foutT.at[g, pl.ds(row0, 16), pl.ds(c * CHUNK, CHUNK)])
                return cc
            lax.fori_loop(0, NCH, chunk_body, 0)

            pltpu.sync_copy(hT.at[pl.ds(tb, TBL)], tbl_a)

            @plsc.parallel_loop(0, EPG // 16)
            def p2_body(gi):
                base = gi * 16
                src_v = src_sp[pl.ds(base, 16)]
                dst_v = dst_sp[pl.ds(base, 16)]
                dacc = [zero16] * 4
                for b in range(16):
                    dacc[b % 4] = dacc[b % 4] + plsc.load_gather(
                        den16_sp, [dst_v + b * NPAD])
                den_v = (dacc[0] + dacc[1]) + (dacc[2] + dacc[3])
                a_v = ee_sp[pl.ds(base, 16)] / den_v
                for l in range(16):
                    src_b = _bcast(src_v, l)
                    dst_b = _bcast(dst_v, l)
                    a_b = _bcast(a_v, l)
                    hv = plsc.load_gather(tbl_a, [src_b + lane_bank])
                    plsc.addupdate_scatter(
                        hacc_sp, [dst_b + lane_bank], hv * a_b)
            pltpu.sync_copy(hacc_sp, haccT.at[pl.ds(tb, TBL)])
        return carry

    lax.fori_loop(0, UPW, unit_body, 0)


def kernel(node_x, edge_x, Radd, params, src, dst, node_gid, edge_gid):
    p1, p2 = params['egat1'], params['egat2']
    off = (jnp.arange(NE, dtype=_i32) // EPG) * NPG
    src_l = src - off
    dst_l = dst - off

    xT = node_x.reshape(NG, NPG, -1).transpose(0, 2, 1)
    xT = jnp.pad(xT, ((0, 0), (0, 0), (0, NPAD - NPG)))
    eT = edge_x.reshape(NG, EPG, -1).transpose(0, 2, 1)

    feats_n, feats_e = xT, eT
    for p in (p1, p2, p2):
        ni, nj, hh = _node_proj(feats_n, p['W_ni'].T, p['W_nj'].T,
                                p['W_node'].T)
        fij = _edge_proj(feats_e, p['W_fij'].T, p['bias'][:, None])
        foutT, hacc_flat = _sc_egat(
            ni.reshape(-1), nj.reshape(-1), hh.reshape(-1), fij,
            p['attn'].reshape(-1), src_l, dst_l)
        feats_n, feats_e = hacc_flat.reshape(NG, D, NPAD), foutT

    Gn = _readout(feats_n, params['aggN_W'].T, params['aggN_b'][:, None],
                  NPAD, mask_cols=NPG)
    Ge = _readout(feats_e, params['aggE_W'].T, params['aggE_b'][:, None], EPG)
    G = jnp.concatenate([Gn.reshape(NG, D), Ge.reshape(NG, D), Radd], axis=1)
    out = pl.pallas_call(
        _mlp_body,
        out_shape=jax.ShapeDtypeStruct((NG, 1), _f32),
    )(G, params['mlp1_W'], params['mlp1_b'], params['mlp2_W'],
      params['mlp2_b'], params['mlp3_W'], params['mlp3_b'])
    return out
